# baseline (speedup 1.0000x reference)
import jax
import jax.numpy as jnp
from jax import lax
from jax.experimental import pallas as pl
from jax.experimental.pallas import tpu as pltpu
from jax.experimental.pallas import tpu_sc as plsc

B = 128
D = 2048
H = D // 2
CACHE = 16384

NB = 3
NC = 32
RB = CACHE // NC

SC_NW = 32
SC_ROWS = 9216
SC_RP = SC_ROWS // SC_NW
SC_CH = 32
SC_NCH = SC_RP // SC_CH
TAIL_NC = (CACHE - SC_ROWS) // 512


def _sc_copy_body(lf2_hbm, new2_hbm, b0, b1, rs0, rs1, ws0, ws1):
    wid = lax.axis_index("s") * 2 + lax.axis_index("c")
    base = wid * SC_RP
    bufs = (b0, b1)
    rsem = (rs0, rs1)
    wsem = (ws0, ws1)

    def rd(k, i):
        return pltpu.async_copy(
            lf2_hbm.at[pl.ds(base + k * SC_CH, SC_CH)], bufs[i], rsem[i])

    def wr(k, i):
        return pltpu.async_copy(
            bufs[i], new2_hbm.at[pl.ds(base + k * SC_CH, SC_CH)], wsem[i])

    h_r = [None] * SC_NCH
    h_w = [None] * SC_NCH
    h_r[0] = rd(0, 0)
    for k in range(SC_NCH):
        cur, oth = k % 2, (k + 1) % 2
        h_r[k].wait()
        if k + 1 < SC_NCH:
            if k >= 1:
                h_w[k - 1].wait()
            h_r[k + 1] = rd(k + 1, oth)
        h_w[k] = wr(k, cur)
    h_w[SC_NCH - 1].wait()
    if SC_NCH >= 2:
        h_w[SC_NCH - 2].wait()


def _lf_kernel(x_ref, pre_ref, out_idx_ref, idx_row_ref, idx_col_ref,
               b1_ref, b2_ref, nw_ref,
               w1_hbm, w2_hbm, lf1_ref, lf2_ref,
               out_ref, new1_ref, v2_out_ref,
               w1_ref, w2_ref, g1_ref, g2_ref, v1_ref, out1_ref,
               buf1, r1, w1s, gsem, wa_sem, wb_sem, ssem):
    def rd1(k, b):
        return pltpu.make_async_copy(lf1_ref.at[pl.ds(k * RB, RB)],
                                     buf1.at[b], r1.at[b])

    def wr1(k, b):
        return pltpu.make_async_copy(buf1.at[b],
                                     new1_ref.at[pl.ds(k * RB, RB)], w1s.at[b])

    pltpu.make_async_copy(w1_hbm, w1_ref, wa_sem).start()
    pltpu.make_async_copy(w2_hbm, w2_ref, wb_sem).start()
    for b in range(NB):
        rd1(b, b).start()

    def gather_start(i, _):
        j = pre_ref[0, i]
        pltpu.make_async_copy(lf1_ref.at[j], g1_ref.at[i], gsem).start()
        pltpu.make_async_copy(lf2_ref.at[j], g2_ref.at[i], gsem).start()
        return 0

    jax.lax.fori_loop(0, B, gather_start, 0)

    SL1 = 4
    SL2 = 8
    C1 = H // SL1
    C2 = D // SL2

    def piece_gather_wait():
        def gather_wait(i, _):
            j = pre_ref[0, i]
            pltpu.make_async_copy(lf1_ref.at[j], g1_ref.at[i], gsem).wait()
            pltpu.make_async_copy(lf2_ref.at[j], g2_ref.at[i], gsem).wait()
            return 0
        jax.lax.fori_loop(0, B, gather_wait, 0)

    def _pmat():
        col = idx_col_ref[...]
        row = idx_row_ref[...]
        eq = col == row
        jj = jax.lax.broadcasted_iota(jnp.int32, (B, B), 1)
        last = jnp.max(jnp.where(eq, jj, -1), axis=1, keepdims=True)
        return (jj == last).astype(jnp.float32)

    def piece_p():
        v1_ref[...] = jnp.dot(_pmat(), x_ref[...],
                              preferred_element_type=jnp.float32)

    def piece_scat1_start():
        def s1(i, _):
            pltpu.make_async_copy(v1_ref.at[i],
                                  new1_ref.at[out_idx_ref[0, i]],
                                  ssem).start()
            return 0
        jax.lax.fori_loop(0, B, s1, 0)

    def piece_w1_wait():
        pltpu.make_async_copy(w1_hbm, w1_ref, wa_sem).wait()

    def piece_stage1(s):
        c = pl.ds(s * C1, C1)
        cb = pl.ds(H + s * C1, C1)
        out1_ref[:, c] = (
            jnp.dot(g1_ref[...], w1_ref[:, c],
                    preferred_element_type=jnp.float32)
            + jnp.dot(x_ref[...], w1_ref[:, cb],
                      preferred_element_type=jnp.float32)
            + b1_ref[:, c])

    def piece_v2():
        v2_out_ref[...] = jnp.dot(_pmat(), out1_ref[...],
                                  preferred_element_type=jnp.float32)

    def piece_w2_wait():
        pltpu.make_async_copy(w2_hbm, w2_ref, wb_sem).wait()

    def piece_stage2(s):
        c = pl.ds(s * C2, C2)
        cb = pl.ds(D + s * C2, C2)
        out_ref[:, c] = (
            jnp.dot(g2_ref[...], w2_ref[:, c],
                    preferred_element_type=jnp.float32)
            + jnp.dot(out1_ref[...], w2_ref[:, cb],
                      preferred_element_type=jnp.float32)
            + b2_ref[:, c])

    def piece_norm():
        out3 = out_ref[...] + x_ref[...]
        var = jnp.mean(out3 * out3, axis=-1, keepdims=True)
        out_ref[...] = out3 * jax.lax.rsqrt(var + 1e-6) * nw_ref[...]

    pieces = ([piece_gather_wait, piece_p, piece_w1_wait]
              + [lambda s=s: piece_stage1(s) for s in range(SL1)]
              + [piece_v2, piece_w2_wait]
              + [lambda s=s: piece_stage2(s) for s in range(SL2)]
              + [piece_norm])
    first_piece_at = 4

    pc = 0
    for j in range(NC):
        if j >= 1 and j - 1 + NB < NC:
            pk = j - 1
            nk = pk + NB
            wr1(pk, pk % NB).wait()
            rd1(nk, nk % NB).start()
        b = j % NB
        rd1(j, b).wait()
        wr1(j, b).start()
        if j >= first_piece_at and pc < len(pieces):
            pieces[pc]()
            pc += 1
    while pc < len(pieces):
        pieces[pc]()
        pc += 1
    for j in range(max(0, NC - NB), NC):
        wr1(j, j % NB).wait()

    piece_scat1_start()

    def scat_wait(i, _):
        pltpu.make_async_copy(v1_ref.at[i], new1_ref.at[out_idx_ref[0, i]],
                              ssem).wait()
        return 0

    jax.lax.fori_loop(0, B, scat_wait, 0)


def _scat2_kernel(out_idx_ref, v2_ref, lf2_ref, new2in_ref, new2_ref,
                  tbuf, tr, tw, ssem):
    del new2in_ref
    TNB = 3

    def trd(k, b):
        return pltpu.make_async_copy(
            lf2_ref.at[pl.ds(SC_ROWS + k * 512, 512)], tbuf.at[b], tr.at[b])

    def twr(k, b):
        return pltpu.make_async_copy(
            tbuf.at[b], new2_ref.at[pl.ds(SC_ROWS + k * 512, 512)], tw.at[b])

    for b in range(min(TNB, TAIL_NC)):
        trd(b, b).start()
    for j in range(TAIL_NC):
        if j >= 1 and j - 1 + TNB < TAIL_NC:
            pk = j - 1
            nk = pk + TNB
            twr(pk, pk % TNB).wait()
            trd(nk, nk % TNB).start()
        b = j % TNB
        trd(j, b).wait()
        twr(j, b).start()
    for j in range(max(0, TAIL_NC - TNB), TAIL_NC):
        twr(j, j % TNB).wait()

    def s2(i, _):
        pltpu.make_async_copy(v2_ref.at[i], new2_ref.at[out_idx_ref[0, i]],
                              ssem).start()
        return 0

    jax.lax.fori_loop(0, B, s2, 0)

    def s2w(i, _):
        pltpu.make_async_copy(v2_ref.at[i], new2_ref.at[out_idx_ref[0, i]],
                              ssem).wait()
        return 0

    jax.lax.fori_loop(0, B, s2w, 0)


def kernel(inputs, pre_lf_indexs, out_lf_indexs, input_lf_loc, out_lf_loc,
           inputs_loc, outputs_loc, kv_cache, conv1_weight, conv1_bias,
           conv2_weight, conv2_bias, lf1_caches, lf2_caches, norm_weight):
    pre_i32 = pre_lf_indexs.astype(jnp.int32)
    out_i32 = out_lf_indexs.astype(jnp.int32)
    pre_sm = pre_i32.reshape(1, B)
    out_sm = out_i32.reshape(1, B)
    idx_row = out_i32.reshape(1, B)
    idx_col = out_i32.reshape(B, 1)

    vmem = pl.BlockSpec(memory_space=pltpu.MemorySpace.VMEM)
    smem = pl.BlockSpec(memory_space=pltpu.MemorySpace.SMEM)
    anym = pl.BlockSpec(memory_space=pl.ANY)

    sc_copy = pl.kernel(
        _sc_copy_body,
        out_type=jax.ShapeDtypeStruct((CACHE, H), jnp.float32),
        mesh=plsc.VectorSubcoreMesh(core_axis_name="c", subcore_axis_name="s"),
        scratch_types=[
            pltpu.VMEM((SC_CH, H), jnp.float32),
            pltpu.VMEM((SC_CH, H), jnp.float32),
            pltpu.SemaphoreType.DMA,
            pltpu.SemaphoreType.DMA,
            pltpu.SemaphoreType.DMA,
            pltpu.SemaphoreType.DMA,
        ],
    )
    new2_bulk = sc_copy(lf2_caches)

    out, new1, v2 = pl.pallas_call(
        _lf_kernel,
        out_shape=[
            jax.ShapeDtypeStruct((B, D), jnp.float32),
            jax.ShapeDtypeStruct((CACHE, D), jnp.float32),
            jax.ShapeDtypeStruct((B, H), jnp.float32),
        ],
        in_specs=[vmem, smem, smem, vmem, vmem,
                  vmem, vmem, vmem,
                  anym, anym, anym, anym],
        out_specs=[vmem, anym, vmem],
        scratch_shapes=[
            pltpu.VMEM((D, D), jnp.float32),
            pltpu.VMEM((H, 2 * D), jnp.float32),
            pltpu.VMEM((B, D), jnp.float32),
            pltpu.VMEM((B, H), jnp.float32),
            pltpu.VMEM((B, D), jnp.float32),
            pltpu.VMEM((B, H), jnp.float32),
            pltpu.VMEM((NB, RB, D), jnp.float32),
            pltpu.SemaphoreType.DMA((NB,)),
            pltpu.SemaphoreType.DMA((NB,)),
            pltpu.SemaphoreType.DMA,
            pltpu.SemaphoreType.DMA,
            pltpu.SemaphoreType.DMA,
            pltpu.SemaphoreType.DMA,
        ],
        compiler_params=pltpu.CompilerParams(
            vmem_limit_bytes=110 * 1024 * 1024,
        ),
    )(inputs, pre_sm, out_sm, idx_row, idx_col,
      conv1_bias.reshape(1, H), conv2_bias.reshape(1, D),
      norm_weight.reshape(1, D),
      conv1_weight, conv2_weight, lf1_caches, lf2_caches)

    new2 = pl.pallas_call(
        _scat2_kernel,
        out_shape=jax.ShapeDtypeStruct((CACHE, H), jnp.float32),
        in_specs=[smem, vmem, anym, anym],
        out_specs=anym,
        scratch_shapes=[
            pltpu.VMEM((3, 512, H), jnp.float32),
            pltpu.SemaphoreType.DMA((3,)),
            pltpu.SemaphoreType.DMA((3,)),
            pltpu.SemaphoreType.DMA,
        ],
        input_output_aliases={3: 0},
        compiler_params=pltpu.CompilerParams(
            vmem_limit_bytes=110 * 1024 * 1024,
        ),
    )(out_sm, v2, lf2_caches, new2_bulk)

    return out, new1, new2


# speedup vs baseline: 1.2001x; 1.2001x over previous
import jax
import jax.numpy as jnp
from jax.experimental import pallas as pl
from jax.experimental.pallas import tpu as pltpu

B = 128
D = 2048
H = D // 2
CACHE = 16384

NB = 2
NC = 16
RB = CACHE // NC


def _lf_kernel(x_ref, pre_ref, out_idx_ref, idx_row_ref, idx_col_ref,
               b1_ref, b2_ref, nw_ref,
               w1_hbm, w2_hbm, lf1_ref, lf2_ref,
               out_ref, new1_ref, new2_ref,
               w1_ref, w2_ref, g1_ref, g2_ref, v1_ref, v2_ref, out1_ref,
               buf1, buf2, r1, w1s, r2, w2s, gsem, wa_sem, wb_sem, ssem):
    def rd1(k, b):
        return pltpu.make_async_copy(lf1_ref.at[pl.ds(k * RB, RB)],
                                     buf1.at[b], r1.at[b])

    def wr1(k, b):
        return pltpu.make_async_copy(buf1.at[b],
                                     new1_ref.at[pl.ds(k * RB, RB)], w1s.at[b])

    def rd2(k, b):
        return pltpu.make_async_copy(lf2_ref.at[pl.ds(k * RB, RB)],
                                     buf2.at[b], r2.at[b])

    def wr2(k, b):
        return pltpu.make_async_copy(buf2.at[b],
                                     new2_ref.at[pl.ds(k * RB, RB)], w2s.at[b])

    pltpu.make_async_copy(w1_hbm, w1_ref, wa_sem).start()
    pltpu.make_async_copy(w2_hbm, w2_ref, wb_sem).start()
    for b in range(NB):
        rd1(b, b).start()
        rd2(b, b).start()

    SL1 = 4
    SL2 = 8
    C1 = H // SL1
    C2 = D // SL2

    def piece_gather_start(k):
        def gather_start(i, _):
            j = pre_ref[0, i]
            pltpu.make_async_copy(lf1_ref.at[j], g1_ref.at[i], gsem).start()
            pltpu.make_async_copy(lf2_ref.at[j], g2_ref.at[i], gsem).start()
            return 0
        jax.lax.fori_loop(k * 64, (k + 1) * 64, gather_start, 0)

    def piece_gather_wait():
        def gather_wait(i, _):
            j = pre_ref[0, i]
            pltpu.make_async_copy(lf1_ref.at[j], g1_ref.at[i], gsem).wait()
            pltpu.make_async_copy(lf2_ref.at[j], g2_ref.at[i], gsem).wait()
            return 0
        jax.lax.fori_loop(0, B, gather_wait, 0)

    def piece_p():
        col = idx_col_ref[...]
        row = idx_row_ref[...]
        eq = col == row
        jj = jax.lax.broadcasted_iota(jnp.int32, (B, B), 1)
        last = jnp.max(jnp.where(eq, jj, -1), axis=1, keepdims=True)
        p = (jj == last).astype(jnp.float32)
        v1_ref[...] = jnp.dot(p, x_ref[...], preferred_element_type=jnp.float32)

    def piece_w1_wait():
        pltpu.make_async_copy(w1_hbm, w1_ref, wa_sem).wait()

    def piece_stage1(s):
        c = pl.ds(s * C1, C1)
        cb = pl.ds(H + s * C1, C1)
        out1_ref[:, c] = (
            jnp.dot(g1_ref[...], w1_ref[:, c],
                    preferred_element_type=jnp.float32)
            + jnp.dot(x_ref[...], w1_ref[:, cb],
                      preferred_element_type=jnp.float32)
            + b1_ref[:, c])

    def piece_v2():
        col = idx_col_ref[...]
        row = idx_row_ref[...]
        eq = col == row
        jj = jax.lax.broadcasted_iota(jnp.int32, (B, B), 1)
        last = jnp.max(jnp.where(eq, jj, -1), axis=1, keepdims=True)
        p = (jj == last).astype(jnp.float32)
        v2_ref[...] = jnp.dot(p, out1_ref[...],
                              preferred_element_type=jnp.float32)

    def piece_w2_wait():
        pltpu.make_async_copy(w2_hbm, w2_ref, wb_sem).wait()

    def piece_stage2(s):
        c = pl.ds(s * C2, C2)
        cb = pl.ds(D + s * C2, C2)
        out_ref[:, c] = (
            jnp.dot(g2_ref[...], w2_ref[:, c],
                    preferred_element_type=jnp.float32)
            + jnp.dot(out1_ref[...], w2_ref[:, cb],
                      preferred_element_type=jnp.float32)
            + b2_ref[:, c])

    def piece_norm():
        out3 = out_ref[...] + x_ref[...]
        var = jnp.mean(out3 * out3, axis=-1, keepdims=True)
        out_ref[...] = out3 * jax.lax.rsqrt(var + 1e-6) * nw_ref[...]

    pieces = ([lambda k=k: piece_gather_start(k) for k in range(2)]
              + [piece_gather_wait, piece_p, piece_w1_wait]
              + [lambda s=s: piece_stage1(s) for s in range(SL1)]
              + [piece_v2, piece_w2_wait]
              + [lambda s=s: piece_stage2(s) for s in range(SL2)]
              + [piece_norm])
    first_piece_at = 0

    pc = 0
    for j in range(NC):
        if j >= 1 and j - 1 + NB < NC:
            pk = j - 1
            nk = pk + NB
            wr1(pk, pk % NB).wait()
            rd1(nk, nk % NB).start()
            wr2(pk, pk % NB).wait()
            rd2(nk, nk % NB).start()
        b = j % NB
        rd1(j, b).wait()
        wr1(j, b).start()
        rd2(j, b).wait()
        wr2(j, b).start()
        if j >= first_piece_at and pc < len(pieces):
            pieces[pc]()
            pc += 1
    while pc < len(pieces):
        pieces[pc]()
        pc += 1
    for j in range(max(0, NC - NB), NC):
        wr1(j, j % NB).wait()
        wr2(j, j % NB).wait()

    def scat_start(i, _):
        k = out_idx_ref[0, i]
        pltpu.make_async_copy(v1_ref.at[i], new1_ref.at[k], ssem).start()
        pltpu.make_async_copy(v2_ref.at[i], new2_ref.at[k], ssem).start()
        return 0

    jax.lax.fori_loop(0, B, scat_start, 0)

    def scat_wait(i, _):
        k = out_idx_ref[0, i]
        pltpu.make_async_copy(v1_ref.at[i], new1_ref.at[k], ssem).wait()
        pltpu.make_async_copy(v2_ref.at[i], new2_ref.at[k], ssem).wait()
        return 0

    jax.lax.fori_loop(0, B, scat_wait, 0)


def kernel(inputs, pre_lf_indexs, out_lf_indexs, input_lf_loc, out_lf_loc,
           inputs_loc, outputs_loc, kv_cache, conv1_weight, conv1_bias,
           conv2_weight, conv2_bias, lf1_caches, lf2_caches, norm_weight):
    pre_i32 = pre_lf_indexs.astype(jnp.int32)
    out_i32 = out_lf_indexs.astype(jnp.int32)
    pre_sm = pre_i32.reshape(1, B)
    out_sm = out_i32.reshape(1, B)
    idx_row = out_i32.reshape(1, B)
    idx_col = out_i32.reshape(B, 1)

    vmem = pl.BlockSpec(memory_space=pltpu.MemorySpace.VMEM)
    smem = pl.BlockSpec(memory_space=pltpu.MemorySpace.SMEM)
    anym = pl.BlockSpec(memory_space=pl.ANY)

    out, new1, new2 = pl.pallas_call(
        _lf_kernel,
        out_shape=[
            jax.ShapeDtypeStruct((B, D), jnp.float32),
            jax.ShapeDtypeStruct((CACHE, D), jnp.float32),
            jax.ShapeDtypeStruct((CACHE, H), jnp.float32),
        ],
        in_specs=[vmem, smem, smem, vmem, vmem,
                  vmem, vmem, vmem,
                  anym, anym, anym, anym],
        out_specs=[vmem, anym, anym],
        scratch_shapes=[
            pltpu.VMEM((D, D), jnp.float32),
            pltpu.VMEM((H, 2 * D), jnp.float32),
            pltpu.VMEM((B, D), jnp.float32),
            pltpu.VMEM((B, H), jnp.float32),
            pltpu.VMEM((B, D), jnp.float32),
            pltpu.VMEM((B, H), jnp.float32),
            pltpu.VMEM((B, H), jnp.float32),
            pltpu.VMEM((NB, RB, D), jnp.float32),
            pltpu.VMEM((NB, RB, H), jnp.float32),
            pltpu.SemaphoreType.DMA((NB,)),
            pltpu.SemaphoreType.DMA((NB,)),
            pltpu.SemaphoreType.DMA((NB,)),
            pltpu.SemaphoreType.DMA((NB,)),
            pltpu.SemaphoreType.DMA,
            pltpu.SemaphoreType.DMA,
            pltpu.SemaphoreType.DMA,
            pltpu.SemaphoreType.DMA,
        ],
        compiler_params=pltpu.CompilerParams(
            vmem_limit_bytes=110 * 1024 * 1024,
        ),
    )(inputs, pre_sm, out_sm, idx_row, idx_col,
      conv1_bias.reshape(1, H), conv2_bias.reshape(1, D),
      norm_weight.reshape(1, D),
      conv1_weight, conv2_weight, lf1_caches, lf2_caches)

    return out, new1, new2
